# baseline (speedup 1.0000x reference)
import jax
import jax.numpy as jnp
from jax import lax
from jax.experimental import pallas as pl
from jax.experimental.pallas import tpu as pltpu, tpu_sc as plsc

_B = 16384
_D = 64
_NC = 2
_NS = 16
_NW = _NC * _NS
_BPW = _B // _NW
_CHUNK = 128
_NCHUNK = _BPW // _CHUNK
_NGATHER = 6
_NBUF = 8

_mesh = plsc.VectorSubcoreMesh(
    core_axis_name="c", subcore_axis_name="s",
    num_cores=_NC, num_subcores=_NS,
)


def _body(user_hbm, track_hbm, xu, xtp, xtn, xup, xun, xta,
          u_out, tp_out, tn_out, up_out, un_out, ta_out,
          idx_v, rows_v, cols_v, sem):
    wid = lax.axis_index("s") * _NC + lax.axis_index("c")
    base = wid * _BPW

    tables = (user_hbm, track_hbm, track_hbm, user_hbm, user_hbm, track_hbm)
    idx_in = (xu, xtp, xtn, xup, xun, xta)
    outs = (u_out, tp_out, tn_out, up_out, un_out, ta_out)

    for g in range(_NGATHER):
        pltpu.sync_copy(idx_in[g].at[pl.ds(base, _BPW)], idx_v.at[g])

    lanes = lax.iota(jnp.int32, 16)

    def fire(g, slot):
        for c in range(_NCHUNK):
            pltpu.async_copy(
                tables[g].at[idx_v.at[g, pl.ds(c * _CHUNK, _CHUNK)]],
                rows_v.at[slot * _NCHUNK + c], sem)

    fire(0, 0)
    for g in range(_NGATHER):
        if g + 1 < _NGATHER:
            fire(g + 1, (g + 1) % 2)
        out = outs[g]

        def do_chunk(c):
            buf = (g % 2) * _NCHUNK + c
            pltpu.make_async_copy(
                tables[g].at[idx_v.at[g, pl.ds(0, _CHUNK)]],
                rows_v.at[0], sem).wait()

            def do_rowblock(rb):
                for jj in range(4):
                    j = rb * 4 + jj
                    for b in range(_D // 16):
                        v = rows_v[buf, j, pl.ds(b * 16, 16)]
                        plsc.store_scatter(
                            cols_v,
                            [b * 16 + lanes, jnp.full((16,), j, jnp.int32)], v)

            pl.loop(0, _CHUNK // 4)(do_rowblock)
            pltpu.sync_copy(
                cols_v, out.at[:, pl.ds(base + c * _CHUNK, _CHUNK)])

        pl.loop(0, _NCHUNK)(do_chunk)


_out_struct = jax.ShapeDtypeStruct((_D, _B), jnp.float32)

_gather6 = pl.kernel(
    _body,
    out_type=(_out_struct,) * _NGATHER,
    mesh=_mesh,
    scratch_types=(
        pltpu.VMEM((_NGATHER, _BPW), jnp.int32),
        pltpu.VMEM((_NBUF, _CHUNK, _D), jnp.float32),
        pltpu.VMEM((_D, _CHUNK), jnp.float32),
        pltpu.SemaphoreType.DMA,
    ),
    compiler_params=pltpu.CompilerParams(
        use_tc_tiling_on_sc=False, needs_layout_passes=False),
)


def kernel(user_mat, track_mat, x_user, x_track_pos, x_track_neg,
           x_user_pos, x_user_neg, x_track_anchor):
    outs = _gather6(
        user_mat, track_mat,
        x_user.astype(jnp.int32), x_track_pos.astype(jnp.int32),
        x_track_neg.astype(jnp.int32), x_user_pos.astype(jnp.int32),
        x_user_neg.astype(jnp.int32), x_track_anchor.astype(jnp.int32),
    )
    return tuple(o.T for o in outs)
